# baseline (speedup 1.0000x reference)
import functools

import jax
import jax.numpy as jnp
from jax import lax
from jax.experimental import pallas as pl
from jax.experimental.pallas import tpu as pltpu
from jax.experimental.pallas import tpu_sc as plsc

B, C, D, H, W = 4, 128, 2, 200, 176
NNZ = 60000
CD = C * D
HW = H * W
DHW = D * HW
MAP_PAD = DHW + 16
OUT_FLAT = B * CD * HW
EMPTY = NNZ
VALS_PAD = NNZ + 16

CHUNK = 6000
NCHUNK = NNZ // CHUNK
WUNROLL = 5
WGRP = CHUNK // (16 * WUNROLL)

SEC = 8800
NSEC = DHW // SEC
DUNROLL = 10
DGRP = SEC // (16 * DUNROLL)

CPW = C // 32

TR_ROWS = 8576
NNZ_PAD = 7 * TR_ROWS

_mesh = plsc.VectorSubcoreMesh(core_axis_name="c", subcore_axis_name="s")


def _tr_body(x_ref, o_ref):
    o_ref[...] = x_ref[...].T


def _transpose(features):
    return pl.pallas_call(
        _tr_body,
        grid=(NNZ_PAD // TR_ROWS,),
        in_specs=[pl.BlockSpec((TR_ROWS, C), lambda i: (i, 0))],
        out_specs=pl.BlockSpec((C, TR_ROWS), lambda i: (0, i)),
        out_shape=jax.ShapeDtypeStruct((C, NNZ_PAD), jnp.float32),
    )(features)


@functools.partial(
    pl.kernel,
    out_type=jax.ShapeDtypeStruct((B * DHW,), jnp.int32),
    mesh=_mesh,
    compiler_params=pltpu.CompilerParams(needs_layout_passes=False),
    scratch_types=[
        pltpu.VMEM((MAP_PAD,), jnp.int32),
        [pltpu.VMEM((CHUNK,), jnp.int32) for _ in range(4)],
        [pltpu.VMEM((CHUNK,), jnp.int32) for _ in range(4)],
        pltpu.SemaphoreType.DMA,
    ],
)
def _winner_kernel(b_hbm, z_hbm, y_hbm, x_hbm, maps_hbm, map_v, buf0, buf1, insem):
    cid = lax.axis_index("c")
    sid = lax.axis_index("s")
    wid = sid * 2 + cid

    @pl.when(wid < B)
    def _():
        empty = jnp.full((16,), EMPTY, jnp.int32)

        @plsc.parallel_loop(0, MAP_PAD, step=16, unroll=8)
        def _zero(i):
            map_v[pl.ds(i, 16)] = empty

        bufs = [buf0, buf1]
        srcs = [b_hbm, z_hbm, y_hbm, x_hbm]

        def issue(k):
            dst = bufs[k % 2]
            return [
                pltpu.async_copy(s.at[pl.ds(k * CHUNK, CHUNK)], d, insem)
                for s, d in zip(srcs, dst)
            ]

        pending = issue(0)
        for k in range(NCHUNK):
            for d in pending:
                d.wait()
            if k + 1 < NCHUNK:
                pending = issue(k + 1)
            bb, zz, yy, xx = bufs[k % 2]
            off = k * CHUNK

            def grp(j, carry, bb=bb, zz=zz, yy=yy, xx=xx, off=off):
                for u in range(WUNROLL):
                    lo = j * (16 * WUNROLL) + u * 16
                    vb = bb[pl.ds(lo, 16)]
                    vz = zz[pl.ds(lo, 16)]
                    vy = yy[pl.ds(lo, 16)]
                    vx = xx[pl.ds(lo, 16)]
                    slot = vz * HW + vy * W + vx
                    sidx = jnp.where(vb == wid, slot, DHW)
                    ids = off + lo + lax.iota(jnp.int32, 16)
                    plsc.store_scatter(map_v, [sidx], ids)
                return carry

            lax.fori_loop(0, WGRP, grp, 0)

        pltpu.sync_copy(map_v.at[pl.ds(0, DHW)], maps_hbm.at[pl.ds(wid * DHW, DHW)])


HSEC = 8
NHSEC = H // HSEC
MSEC = HSEC * W
BPC = 4


def _make_densify(b0):
    @functools.partial(
        pl.kernel,
        out_type=jax.ShapeDtypeStruct((BPC, CD, H, W), jnp.float32),
        mesh=_mesh,
        compiler_params=pltpu.CompilerParams(needs_layout_passes=False),
        scratch_types=[
            pltpu.VMEM_SHARED((BPC * DHW,), jnp.int32),
            pltpu.VMEM((VALS_PAD,), jnp.float32),
            [pltpu.VMEM((2 * MSEC,), jnp.int32) for _ in range(4)],
            [pltpu.VMEM((2, HSEC, W), jnp.float32) for _ in range(8)],
            pltpu.SemaphoreType.DMA,
            pltpu.SemaphoreType.DMA,
        ],
    )
    def _densify_kernel(featT_hbm, maps_hbm, out_hbm, maps_sh, vals_v, mbufs,
                        obufs, insem, outsem):
        cid = lax.axis_index("c")
        sid = lax.axis_index("s")
        wid = sid * 2 + cid

        @pl.when(sid == 0)
        def _():
            pltpu.sync_copy(maps_hbm.at[pl.ds(b0 * DHW, BPC * DHW)], maps_sh)

        plsc.subcore_barrier()

        vals_v[pl.ds(NNZ, 16)] = jnp.zeros((16,), jnp.float32)

        NSTEP = BPC * NHSEC

        def c_body(t, carry):
            c = wid * CPW + t
            pltpu.sync_copy(featT_hbm.at[pl.ds(c * NNZ_PAD, NNZ)],
                            vals_v.at[pl.ds(0, NNZ)])

            def in_copy(k, mb):
                bl = k // NHSEC
                s = k - bl * NHSEC
                for z in range(2):
                    off = pl.multiple_of(bl * DHW + z * HW + s * MSEC, 8)
                    pltpu.async_copy(maps_sh.at[pl.ds(off, MSEC)],
                                     mb.at[pl.ds(z * MSEC, MSEC)], insem)

            def wait_in():
                for z in range(2):
                    pltpu.make_async_copy(
                        maps_sh.at[pl.ds(0, MSEC)],
                        mbufs[0].at[pl.ds(z * MSEC, MSEC)], insem).wait()

            def wait_out():
                pltpu.make_async_copy(
                    obufs[0],
                    out_hbm.at[0, pl.ds(0, 2), pl.ds(0, HSEC), :],
                    outsem).wait()

            def compute_and_out(k, mb, ob):
                bl = k // NHSEC
                s = k - bl * NHSEC

                @plsc.parallel_loop(0, HSEC, step=1)
                def _gather(h):
                    for z in range(2):
                        for xg in range(W // 16):
                            lo = z * MSEC + h * W + xg * 16
                            m = mb[pl.ds(lo, 16)]
                            ob[z, h, pl.ds(xg * 16, 16)] = \
                                plsc.load_gather(vals_v, [m])

                srow = pl.multiple_of(s * HSEC, 8)
                pltpu.async_copy(
                    ob,
                    out_hbm.at[bl, pl.ds(2 * c, 2), pl.ds(srow, HSEC), :],
                    outsem)

            in_copy(jnp.int32(0), mbufs[0])
            in_copy(jnp.int32(1), mbufs[1])

            def drain_body(j, c3):
                wait_out()
                return c3

            def step_oct(i, carry2):
                for p in range(8):
                    k = 8 * i + p
                    wait_in()

                    @pl.when(k + 2 < NSTEP)
                    def _(k=k, p=p):
                        in_copy(k + 2, mbufs[(p + 2) % 4])

                    lax.fori_loop(0, jnp.where(i > 0, 1, 0), drain_body, 0)
                    compute_and_out(k, mbufs[p % 4], obufs[p])
                return carry2

            lax.fori_loop(0, NSTEP // 8, step_oct, 0)
            for p in range(NSTEP % 8):
                k = (NSTEP // 8) * 8 + p
                wait_in()
                if k + 2 < NSTEP:
                    in_copy(jnp.int32(k + 2), mbufs[(p + 2) % 4])
                wait_out()
                compute_and_out(jnp.int32(k), mbufs[p % 4], obufs[p])
            lax.fori_loop(0, 8, drain_body, 0)
            return carry

        lax.fori_loop(0, CPW, c_body, 0)

    return _densify_kernel


_densify_all = _make_densify(0)


def kernel(features, b_idx, z_idx, y_idx, x_idx):
    featT = _transpose(features).reshape(-1)
    maps = _winner_kernel(b_idx, z_idx, y_idx, x_idx)
    return _densify_all(featT, maps)


# speedup vs baseline: 1.2207x; 1.2207x over previous
import functools

import jax
import jax.numpy as jnp
from jax import lax
from jax.experimental import pallas as pl
from jax.experimental.pallas import tpu as pltpu
from jax.experimental.pallas import tpu_sc as plsc

B, C, D, H, W = 4, 128, 2, 200, 176
NNZ = 60000
CD = C * D
HW = H * W
DHW = D * HW
MAP_PAD = DHW + 16
OUT_FLAT = B * CD * HW
EMPTY = NNZ
VALS_PAD = NNZ + 16

CHUNK = 6000
NCHUNK = NNZ // CHUNK
WUNROLL = 5
WGRP = CHUNK // (16 * WUNROLL)

SEC = 8800
NSEC = DHW // SEC
DUNROLL = 10
DGRP = SEC // (16 * DUNROLL)

CPW = C // 32

TR_ROWS = 8576
NNZ_PAD = 7 * TR_ROWS

_mesh = plsc.VectorSubcoreMesh(core_axis_name="c", subcore_axis_name="s")


def _tr_body(x_ref, o_ref):
    o_ref[...] = x_ref[...].T


def _transpose(features):
    return pl.pallas_call(
        _tr_body,
        grid=(NNZ_PAD // TR_ROWS,),
        in_specs=[pl.BlockSpec((TR_ROWS, C), lambda i: (i, 0))],
        out_specs=pl.BlockSpec((C, TR_ROWS), lambda i: (0, i)),
        out_shape=jax.ShapeDtypeStruct((C, NNZ_PAD), jnp.float32),
    )(features)


@functools.partial(
    pl.kernel,
    out_type=jax.ShapeDtypeStruct((B * DHW,), jnp.int32),
    mesh=_mesh,
    compiler_params=pltpu.CompilerParams(needs_layout_passes=False),
    scratch_types=[
        pltpu.VMEM((MAP_PAD,), jnp.int32),
        [pltpu.VMEM((CHUNK,), jnp.int32) for _ in range(4)],
        [pltpu.VMEM((CHUNK,), jnp.int32) for _ in range(4)],
        pltpu.SemaphoreType.DMA,
    ],
)
def _winner_kernel(b_hbm, z_hbm, y_hbm, x_hbm, maps_hbm, map_v, buf0, buf1, insem):
    cid = lax.axis_index("c")
    sid = lax.axis_index("s")
    wid = sid * 2 + cid

    @pl.when(wid < B)
    def _():
        empty = jnp.full((16,), EMPTY, jnp.int32)

        @plsc.parallel_loop(0, MAP_PAD, step=16, unroll=8)
        def _zero(i):
            map_v[pl.ds(i, 16)] = empty

        bufs = [buf0, buf1]
        srcs = [b_hbm, z_hbm, y_hbm, x_hbm]

        def issue(k):
            dst = bufs[k % 2]
            return [
                pltpu.async_copy(s.at[pl.ds(k * CHUNK, CHUNK)], d, insem)
                for s, d in zip(srcs, dst)
            ]

        pending = issue(0)
        for k in range(NCHUNK):
            for d in pending:
                d.wait()
            if k + 1 < NCHUNK:
                pending = issue(k + 1)
            bb, zz, yy, xx = bufs[k % 2]
            off = k * CHUNK

            def grp(j, carry, bb=bb, zz=zz, yy=yy, xx=xx, off=off):
                for u in range(WUNROLL):
                    lo = j * (16 * WUNROLL) + u * 16
                    vb = bb[pl.ds(lo, 16)]
                    vz = zz[pl.ds(lo, 16)]
                    vy = yy[pl.ds(lo, 16)]
                    vx = xx[pl.ds(lo, 16)]
                    slot = vz * HW + vy * W + vx
                    sidx = jnp.where(vb == wid, slot, DHW)
                    ids = off + lo + lax.iota(jnp.int32, 16)
                    plsc.store_scatter(map_v, [sidx], ids)
                return carry

            lax.fori_loop(0, WGRP, grp, 0)

        pltpu.sync_copy(map_v.at[pl.ds(0, DHW)], maps_hbm.at[pl.ds(wid * DHW, DHW)])


HSEC = 8
NHSEC = H // HSEC
MSEC = HSEC * W
BPC = 4


def _make_densify(b0):
    @functools.partial(
        pl.kernel,
        out_type=jax.ShapeDtypeStruct((BPC, CD, H, W), jnp.float32),
        mesh=_mesh,
        compiler_params=pltpu.CompilerParams(needs_layout_passes=False),
        scratch_types=[
            pltpu.VMEM_SHARED((BPC * DHW,), jnp.int32),
            pltpu.VMEM((VALS_PAD,), jnp.float32),
            [pltpu.VMEM((2 * MSEC,), jnp.int32) for _ in range(4)],
            [pltpu.VMEM((2, HSEC, W), jnp.float32) for _ in range(4)],
            pltpu.SemaphoreType.DMA,
            pltpu.SemaphoreType.DMA,
        ],
    )
    def _densify_kernel(featT_hbm, maps_hbm, out_hbm, maps_sh, vals_v, mbufs,
                        obufs, insem, outsem):
        cid = lax.axis_index("c")
        sid = lax.axis_index("s")
        wid = sid * 2 + cid

        @pl.when(sid == 0)
        def _():
            pltpu.sync_copy(maps_hbm.at[pl.ds(b0 * DHW, BPC * DHW)], maps_sh)

        plsc.subcore_barrier()

        vals_v[pl.ds(NNZ, 16)] = jnp.zeros((16,), jnp.float32)

        NSTEP = BPC * NHSEC

        def c_body(t, carry):
            c = wid * CPW + t
            pltpu.sync_copy(featT_hbm.at[pl.ds(c * NNZ_PAD, NNZ)],
                            vals_v.at[pl.ds(0, NNZ)])

            def in_copy(k, mb):
                bl = k // NHSEC
                s = k - bl * NHSEC
                for z in range(2):
                    off = pl.multiple_of(bl * DHW + z * HW + s * MSEC, 8)
                    pltpu.async_copy(maps_sh.at[pl.ds(off, MSEC)],
                                     mb.at[pl.ds(z * MSEC, MSEC)], insem)

            def wait_in():
                for z in range(2):
                    pltpu.make_async_copy(
                        maps_sh.at[pl.ds(0, MSEC)],
                        mbufs[0].at[pl.ds(z * MSEC, MSEC)], insem).wait()

            def wait_out():
                pltpu.make_async_copy(
                    obufs[0],
                    out_hbm.at[0, pl.ds(0, 2), pl.ds(0, HSEC), :],
                    outsem).wait()

            def compute_and_out(k, mb, ob):
                bl = k // NHSEC
                s = k - bl * NHSEC

                @plsc.parallel_loop(0, HSEC, step=1)
                def _gather(h):
                    for z in range(2):
                        for xg in range(W // 16):
                            lo = z * MSEC + h * W + xg * 16
                            m = mb[pl.ds(lo, 16)]
                            ob[z, h, pl.ds(xg * 16, 16)] = \
                                plsc.load_gather(vals_v, [m])

                srow = pl.multiple_of(s * HSEC, 8)
                pltpu.async_copy(
                    ob,
                    out_hbm.at[bl, pl.ds(2 * c, 2), pl.ds(srow, HSEC), :],
                    outsem)

            in_copy(jnp.int32(0), mbufs[0])
            in_copy(jnp.int32(1), mbufs[1])

            def drain_body(j, c3):
                wait_out()
                return c3

            def step_quad(i, carry2):
                for p in range(4):
                    k = 4 * i + p
                    wait_in()

                    @pl.when(k + 2 < NSTEP)
                    def _(k=k, p=p):
                        in_copy(k + 2, mbufs[(p + 2) % 4])

                    lax.fori_loop(0, jnp.where(i > 0, 1, 0), drain_body, 0)
                    compute_and_out(k, mbufs[p], obufs[p])
                return carry2

            lax.fori_loop(0, NSTEP // 4, step_quad, 0)
            lax.fori_loop(0, 4, drain_body, 0)
            return carry

        lax.fori_loop(0, CPW, c_body, 0)

    return _densify_kernel


_densify_all = _make_densify(0)


def kernel(features, b_idx, z_idx, y_idx, x_idx):
    featT = _transpose(features).reshape(-1)
    maps = _winner_kernel(b_idx, z_idx, y_idx, x_idx)
    return _densify_all(featT, maps)
